# Initial kernel scaffold; baseline (speedup 1.0000x reference)
import jax
import jax.numpy as jnp
from jax.experimental import pallas as pl


def kernel(node_features, edge_index, node_mask, edge_mask, emb0_Ws, emb0_Wn, emb0_b, emb1_Ws, emb1_Wn, emb1_b, asn0_Ws, asn0_Wn, asn0_b, asn1_Ws, asn1_Wn, asn1_b, lp_W1, lp_b1, lp_W2, lp_b2):
    raise NotImplementedError("write your pallas kernel here")


# speedup vs baseline: 10.3097x; 10.3097x over previous
import jax
import jax.numpy as jnp
from jax import lax
from jax.experimental import pallas as pl
from jax.experimental.pallas import tpu as pltpu
from jax.experimental.pallas import tpu_sc as plsc

B, N, E, DIN, DH, DOUT, C = 8, 1024, 16384, 256, 256, 256, 64
NC, NS, L = 2, 16, 16
GPC = B // NC
CH = 128
NCH = E // NS // CH
NN = N * N
CC = C * C
ET = E // NS
SLOTT = NN // NS
XCH = 64
XCAP = E // XCH
EB = 2048

_f32 = jnp.float32
_i32 = jnp.int32

_mesh = plsc.VectorSubcoreMesh(
    core_axis_name="c", subcore_axis_name="s", num_cores=NC, num_subcores=NS)
_params = pltpu.CompilerParams(needs_layout_passes=False)


def _sc1a_body(srcf, tgtf, zi,
               slot_o,
               sall, tall, fidx, eidv, zbi):
  cid = lax.axis_index("c")
  sid = lax.axis_index("s")
  pltpu.sync_copy(zi, zbi)
  for gi in range(GPC):
    g = cid * GPC + gi
    for q in range(SLOTT // 4096):
      pltpu.sync_copy(
          zbi, slot_o.at[pl.ds(g * NN + sid * SLOTT + q * 4096, 4096)])
  plsc.subcore_barrier()
  for gi in range(GPC):
    g = cid * GPC + gi
    ebase = g * E + sid * ET
    pltpu.sync_copy(srcf.at[pl.ds(ebase, ET)], sall)
    pltpu.sync_copy(tgtf.at[pl.ds(ebase, ET)], tall)
    for j in range(NCH):
      for k in range(CH // L):
        sv = sall[pl.ds(j * CH + k * L, L)]
        tv = tall[pl.ds(j * CH + k * L, L)]
        fidx[pl.ds(k * L, L)] = sv * N + tv + g * NN
        eidv[pl.ds(k * L, L)] = (
            lax.iota(_i32, L) + (sid * ET + j * CH + k * L + 1))
      pltpu.sync_copy(eidv, slot_o.at[fidx])


def _sc1b_body(srcf, tgtf, slot_i,
               exc_o,
               sall, tall, fidx, rbuf, fbuf, excbuf):
  cid = lax.axis_index("c")
  sid = lax.axis_index("s")
  for gi in range(GPC):
    g = cid * GPC + gi
    ebase = g * E + sid * ET
    pltpu.sync_copy(srcf.at[pl.ds(ebase, ET)], sall)
    pltpu.sync_copy(tgtf.at[pl.ds(ebase, ET)], tall)
    for k in range(ET // L):
      excbuf[pl.ds(k * L, L)] = jnp.full((L,), -1, _i32)
    off = _i32(0)
    for j in range(NCH):
      for k in range(CH // L):
        sv = sall[pl.ds(j * CH + k * L, L)]
        tv = tall[pl.ds(j * CH + k * L, L)]
        f = sv * N + tv
        fbuf[pl.ds(k * L, L)] = f
        fidx[pl.ds(k * L, L)] = f + g * NN
      pltpu.sync_copy(slot_i.at[fidx], rbuf)
      for k in range(CH // L):
        rv = rbuf[pl.ds(k * L, L)]
        ev = lax.iota(_i32, L) + (sid * ET + j * CH + k * L + 1)
        m = rv != ev
        mi = m.astype(_i32)
        pos = plsc.cumsum(mi) + (off - 1)
        plsc.store_scatter(excbuf, [pos], fbuf[pl.ds(k * L, L)], mask=m)
        off = off + jnp.sum(mi)
    pltpu.sync_copy(excbuf, exc_o.at[pl.ds((g * NS + sid) * ET, ET)])


def _sc1c_body(x2, exc_i, zf,
               aggx_o, degx_o,
               excall, own2d, tl2d, tlbuf, gbuf, stripe, degst):
  cid = lax.axis_index("c")
  sid = lax.axis_index("s")
  for gi in range(GPC):
    g = cid * GPC + gi
    pltpu.sync_copy(zf, stripe)
    for k in range(4):
      degst[pl.ds(k * L, L)] = jnp.zeros((L,), _f32)
    nmat = _i32(0)

    def scan_body(k, nmat):
      pv = excall[pl.ds(k * L, L)]
      t = jnp.bitwise_and(pv, N - 1)
      m = jnp.logical_and(pv >= 0, lax.shift_right_logical(t, 6) == sid)
      mi = m.astype(_i32)
      pos = plsc.cumsum(mi) + (nmat - 1)
      grow = lax.shift_right_logical(pv, 10) + g * N
      tl = jnp.bitwise_and(pv, 63)
      r = lax.shift_right_logical(pos, 6)
      cc = jnp.bitwise_and(pos, XCH - 1)
      plsc.store_scatter(own2d, [r, cc], grow, mask=m)
      plsc.store_scatter(tl2d, [r, cc], tl, mask=m)
      return nmat + jnp.sum(mi)

    for q in range(NS):
      pltpu.sync_copy(exc_i.at[pl.ds(g * NS * ET + q * ET, ET)], excall)
      nmat = lax.fori_loop(0, ET // L, scan_body, nmat)
    rlast = jnp.minimum(lax.shift_right_logical(nmat, 6), XCAP - 1)
    full_cap = nmat >= XCAP * XCH
    nrem = jnp.bitwise_and(nmat, XCH - 1)
    for k in range(XCH // L):
      v = own2d[rlast, pl.ds(k * L, L)]
      keep = jnp.logical_or((lax.iota(_i32, L) + (k * L)) < nrem, full_cap)
      vv = jnp.where(keep, v, g * N)
      plsc.store_scatter(own2d,
                         [jnp.full((L,), rlast, _i32),
                          lax.iota(_i32, L) + (k * L)], vv)
    nch = lax.shift_right_logical(nmat + (XCH - 1), 6)

    def chunk_body(c, acc):
      pltpu.sync_copy(x2.at[own2d.at[c]], gbuf)
      for k in range(XCH // L):
        tlbuf[pl.ds(k * L, L)] = tl2d[c, pl.ds(k * L, L)]
      nin = jnp.minimum(nmat - c * XCH, XCH)

      def ebody(e, a2):
        tl = jnp.bitwise_and(tlbuf[pl.ds(e, L)][0], 63)
        for k2 in range(DIN // L):
          xv = gbuf[e, pl.ds(k2 * L, L)]
          idxv = tl * DIN + (k2 * L) + lax.iota(_i32, L)
          plsc.addupdate_scatter(stripe, [idxv], xv)
        onemask = lax.iota(_i32, L) == 0
        plsc.addupdate_scatter(degst, [jnp.full((L,), tl, _i32)],
                               jnp.ones((L,), _f32), mask=onemask)
        return a2

      return lax.fori_loop(0, nin, ebody, acc)

    lax.fori_loop(0, nch, chunk_body, _i32(0))
    pltpu.sync_copy(stripe,
                    aggx_o.at[pl.ds(g * N * DIN + sid * 64 * DIN, 64 * DIN)])
    pltpu.sync_copy(degst.at[pl.ds(0, 64)],
                    degx_o.at[pl.ds(g * N + sid * 64, 64)])


def _sc2_body(h2, a2, exc_i, zf,
              aggh_o, agga_o,
              excall, own2d, tl2d, tlbuf, gbuf, strh, stra):
  cid = lax.axis_index("c")
  sid = lax.axis_index("s")
  for gi in range(GPC):
    g = cid * GPC + gi
    pltpu.sync_copy(zf, strh)
    pltpu.sync_copy(zf, stra)
    nmat = _i32(0)

    def scan_body(k, nmat):
      pv = excall[pl.ds(k * L, L)]
      t = jnp.bitwise_and(pv, N - 1)
      m = jnp.logical_and(pv >= 0, lax.shift_right_logical(t, 6) == sid)
      mi = m.astype(_i32)
      pos = plsc.cumsum(mi) + (nmat - 1)
      grow = lax.shift_right_logical(pv, 10) + g * N
      tl = jnp.bitwise_and(pv, 63)
      r = lax.shift_right_logical(pos, 6)
      cc = jnp.bitwise_and(pos, XCH - 1)
      plsc.store_scatter(own2d, [r, cc], grow, mask=m)
      plsc.store_scatter(tl2d, [r, cc], tl, mask=m)
      return nmat + jnp.sum(mi)

    for q in range(NS):
      pltpu.sync_copy(exc_i.at[pl.ds(g * NS * ET + q * ET, ET)], excall)
      nmat = lax.fori_loop(0, ET // L, scan_body, nmat)
    rlast = jnp.minimum(lax.shift_right_logical(nmat, 6), XCAP - 1)
    full_cap = nmat >= XCAP * XCH
    nrem = jnp.bitwise_and(nmat, XCH - 1)
    for k in range(XCH // L):
      v = own2d[rlast, pl.ds(k * L, L)]
      keep = jnp.logical_or((lax.iota(_i32, L) + (k * L)) < nrem, full_cap)
      vv = jnp.where(keep, v, g * N)
      plsc.store_scatter(own2d,
                         [jnp.full((L,), rlast, _i32),
                          lax.iota(_i32, L) + (k * L)], vv)
    nch = lax.shift_right_logical(nmat + (XCH - 1), 6)

    def chunk_body(c, acc):
      for k in range(XCH // L):
        tlbuf[pl.ds(k * L, L)] = tl2d[c, pl.ds(k * L, L)]
      nin = jnp.minimum(nmat - c * XCH, XCH)
      pltpu.sync_copy(h2.at[own2d.at[c]], gbuf)

      def ebody(e, a2c):
        tl = jnp.bitwise_and(tlbuf[pl.ds(e, L)][0], 63)
        for k2 in range(DH // L):
          idxv = tl * DH + (k2 * L) + lax.iota(_i32, L)
          plsc.addupdate_scatter(strh, [idxv], gbuf[e, pl.ds(k2 * L, L)])
        return a2c

      lax.fori_loop(0, nin, ebody, acc)
      pltpu.sync_copy(a2.at[own2d.at[c]], gbuf)

      def ebody2(e, a2c):
        tl = jnp.bitwise_and(tlbuf[pl.ds(e, L)][0], 63)
        for k2 in range(DH // L):
          idxv = tl * DH + (k2 * L) + lax.iota(_i32, L)
          plsc.addupdate_scatter(stra, [idxv], gbuf[e, pl.ds(k2 * L, L)])
        return a2c

      return lax.fori_loop(0, nin, ebody2, acc)

    lax.fori_loop(0, nch, chunk_body, _i32(0))
    pltpu.sync_copy(strh,
                    aggh_o.at[pl.ds(g * N * DH + sid * 64 * DH, 64 * DH)])
    pltpu.sync_copy(stra,
                    agga_o.at[pl.ds(g * N * DH + sid * 64 * DH, 64 * DH)])


def _sc3a_body(u2, v2, gsrcf, gtgtf,
               ug_o, vg_o,
               six, tix, gbuf):
  cid = lax.axis_index("c")
  sid = lax.axis_index("s")
  for gi in range(GPC):
    g = cid * GPC + gi
    ebase = g * E + sid * ET
    for j in range(NCH):
      pltpu.sync_copy(gsrcf.at[pl.ds(ebase + j * CH, CH)], six)
      pltpu.sync_copy(gtgtf.at[pl.ds(ebase + j * CH, CH)], tix)
      pltpu.sync_copy(u2.at[six], gbuf)
      pltpu.sync_copy(gbuf, ug_o.at[pl.ds(ebase + j * CH, CH)])
      pltpu.sync_copy(v2.at[tix], gbuf)
      pltpu.sync_copy(gbuf, vg_o.at[pl.ds(ebase + j * CH, CH)])


def _sc3b_body(par, zi4,
               pei_o, pem_o,
               pav, fbuf, sbuf, tbuf, mbuf):
  cid = lax.axis_index("c")
  sid = lax.axis_index("s")

  @pl.when(sid < GPC)
  def _():
    g = cid * GPC + sid
    pltpu.sync_copy(par.at[pl.ds(g * (CC // L), CC // L)], pav)
    pltpu.sync_copy(zi4, fbuf)
    off = _i32(0)
    for k in range(CC // L):
      pa = pav[k]
      m = pa > 1e-6
      mi = m.astype(_i32)
      fv = lax.iota(_i32, L) + (k * L)
      pos = plsc.cumsum(mi) + (off - 1)
      plsc.store_scatter(fbuf, [pos], fv, mask=m)
      off = off + jnp.sum(mi)
    for k in range(CC // L):
      fv = fbuf[pl.ds(k * L, L)]
      sbuf[pl.ds(k * L, L)] = lax.shift_right_logical(fv, 6)
      tbuf[pl.ds(k * L, L)] = jnp.bitwise_and(fv, C - 1)
      inr = (lax.iota(_i32, L) + (k * L)) < off
      mbuf[pl.ds(k * L, L)] = jnp.where(inr, _f32(1.0), _f32(0.0))
    pltpu.sync_copy(sbuf, pei_o.at[pl.ds((g * 2) * CC, CC)])
    pltpu.sync_copy(tbuf, pei_o.at[pl.ds((g * 2 + 1) * CC, CC)])
    pltpu.sync_copy(mbuf, pem_o.at[pl.ds(g * CC, CC)])


def _tc1_body(x_ref, slot_ref, aggx_ref, degx_ref,
              wse, wne, bbe, wsa, wna, bba,
              h_ref, a0_ref, deg_ref):
  x = x_ref[0]
  adj = (slot_ref[0] > 0).astype(_f32)
  agg = lax.dot_general(adj, x, (((0,), (0,)), ((), ())),
                        preferred_element_type=_f32)
  agg += aggx_ref[0]
  deg = lax.dot_general(adj, jnp.ones((N, 1), _f32), (((0,), (0,)), ((), ())),
                        preferred_element_type=_f32) + degx_ref[0]
  deg_ref[0] = deg
  degc = jnp.maximum(deg, 1e-8)
  aggn = agg / degc
  h = jnp.dot(x, wse[...], preferred_element_type=_f32)
  h += jnp.dot(aggn, wne[...], preferred_element_type=_f32)
  h_ref[0] = jnp.maximum(h + bbe[...], 0.0)
  a = jnp.dot(x, wsa[...], preferred_element_type=_f32)
  a += jnp.dot(aggn, wna[...], preferred_element_type=_f32)
  a0_ref[0] = jnp.maximum(a + bba[...], 0.0)


def _tc2_body(h_ref, a0_ref, slot_ref, aggh_ref, agga_ref, deg_ref,
              wse, wne, bbe, wsa, wna, bba, w1a, w1b,
              pooled_ref, pmask_ref, padj_ref, u_ref, v_ref, ent_ref):
  g = pl.program_id(0)
  adj = (slot_ref[0] > 0).astype(_f32)
  degc = jnp.maximum(deg_ref[0], 1e-8)
  aggh = lax.dot_general(adj, h_ref[0], (((0,), (0,)), ((), ())),
                         preferred_element_type=_f32) + aggh_ref[0]
  agga = lax.dot_general(adj, a0_ref[0], (((0,), (0,)), ((), ())),
                         preferred_element_type=_f32) + agga_ref[0]
  emb = jnp.dot(h_ref[0], wse[...], preferred_element_type=_f32)
  emb += jnp.dot(aggh / degc, wne[...], preferred_element_type=_f32)
  emb = jnp.maximum(emb + bbe[...], 0.0)
  a1 = jnp.dot(a0_ref[0], wsa[...], preferred_element_type=_f32)
  a1 += jnp.dot(agga / degc, wna[...], preferred_element_type=_f32)
  a1 = jnp.maximum(a1 + bba[...], 0.0)
  mx = jnp.max(a1, axis=-1, keepdims=True)
  ex = jnp.exp(a1 - mx)
  S = ex / jnp.sum(ex, axis=-1, keepdims=True)
  pooled_ref[0] = lax.dot_general(S, emb, (((0,), (0,)), ((), ())),
                                  preferred_element_type=_f32)
  colsum = jnp.sum(S, axis=0, keepdims=True)
  pmask_ref[0] = (colsum > 1e-6).astype(_f32)
  t1 = lax.dot_general(S, adj, (((0,), (0,)), ((), ())),
                       preferred_element_type=_f32)
  padj_ref[0] = jnp.dot(t1, S, preferred_element_type=_f32)
  u_ref[0] = jnp.dot(emb, w1a[...], preferred_element_type=_f32)
  v_ref[0] = jnp.dot(emb, w1b[...], preferred_element_type=_f32)
  ent = -jnp.sum(S * jnp.log(jnp.maximum(S, 1e-8)), axis=-1, keepdims=True)

  @pl.when(g == 0)
  def _():
    ent_ref[...] = jnp.zeros((1, 1), _f32)

  ent_ref[...] += jnp.sum(ent).reshape(1, 1) / (N * B)


def _tc3_body(u_ref, v_ref, bb1, w2r, bb2, out_ref):
  i = pl.program_id(0)
  j = pl.program_id(1)
  w = jnp.maximum(u_ref[0] + v_ref[0] + bb1[...], 0.0)
  z = jnp.sum(w * w2r[...], axis=-1, keepdims=True) + bb2[...]
  p = jnp.clip(jax.nn.sigmoid(z), 1e-7, 1.0 - 1e-7)
  loss = jnp.sum(-jnp.log(p))

  @pl.when(jnp.logical_and(i == 0, j == 0))
  def _():
    out_ref[...] = jnp.zeros((1, 1), _f32)

  out_ref[...] += loss.reshape(1, 1) / (E * B)


def _sc1a_call(srcf, tgtf, zi):
  return pl.kernel(
      _sc1a_body,
      out_type=[jax.ShapeDtypeStruct((B * NN,), _i32)],
      mesh=_mesh,
      compiler_params=_params,
      scratch_types=[
          pltpu.VMEM((ET,), _i32),
          pltpu.VMEM((ET,), _i32),
          pltpu.VMEM((CH,), _i32),
          pltpu.VMEM((CH,), _i32),
          pltpu.VMEM((4096,), _i32),
      ],
  )(srcf, tgtf, zi)


def _sc1b_call(srcf, tgtf, slot):
  return pl.kernel(
      _sc1b_body,
      out_type=[jax.ShapeDtypeStruct((B * E,), _i32)],
      mesh=_mesh,
      compiler_params=_params,
      scratch_types=[
          pltpu.VMEM((ET,), _i32),
          pltpu.VMEM((ET,), _i32),
          pltpu.VMEM((CH,), _i32),
          pltpu.VMEM((CH,), _i32),
          pltpu.VMEM((CH,), _i32),
          pltpu.VMEM((ET,), _i32),
      ],
  )(srcf, tgtf, slot)


def _sc1c_call(x2, exc, zf):
  return pl.kernel(
      _sc1c_body,
      out_type=[jax.ShapeDtypeStruct((B * N * DIN,), _f32),
                jax.ShapeDtypeStruct((B * N,), _f32)],
      mesh=_mesh,
      compiler_params=_params,
      scratch_types=[
          pltpu.VMEM((ET,), _i32),
          pltpu.VMEM((XCAP, XCH), _i32),
          pltpu.VMEM((XCAP, XCH), _i32),
          pltpu.VMEM((XCH + L,), _i32),
          pltpu.VMEM((XCH, DIN), _f32),
          pltpu.VMEM((64 * DIN,), _f32),
          pltpu.VMEM((64,), _f32),
      ],
  )(x2, exc, zf)


def _sc2_call(h2, a2, exc, zf):
  return pl.kernel(
      _sc2_body,
      out_type=[jax.ShapeDtypeStruct((B * N * DH,), _f32),
                jax.ShapeDtypeStruct((B * N * DH,), _f32)],
      mesh=_mesh,
      compiler_params=_params,
      scratch_types=[
          pltpu.VMEM((ET,), _i32),
          pltpu.VMEM((XCAP, XCH), _i32),
          pltpu.VMEM((XCAP, XCH), _i32),
          pltpu.VMEM((XCH + L,), _i32),
          pltpu.VMEM((XCH, DH), _f32),
          pltpu.VMEM((64 * DH,), _f32),
          pltpu.VMEM((64 * DH,), _f32),
      ],
  )(h2, a2, exc, zf)


def _sc3a_call(u2, v2, gsrcf, gtgtf):
  return pl.kernel(
      _sc3a_body,
      out_type=[jax.ShapeDtypeStruct((B * E, DH), _f32),
                jax.ShapeDtypeStruct((B * E, DH), _f32)],
      mesh=_mesh,
      compiler_params=_params,
      scratch_types=[
          pltpu.VMEM((CH,), _i32),
          pltpu.VMEM((CH,), _i32),
          pltpu.VMEM((CH, DH), _f32),
      ],
  )(u2, v2, gsrcf, gtgtf)


def _sc3b_call(par, zi4):
  return pl.kernel(
      _sc3b_body,
      out_type=[jax.ShapeDtypeStruct((B * 2 * CC,), _i32),
                jax.ShapeDtypeStruct((B * CC,), _f32)],
      mesh=_mesh,
      compiler_params=_params,
      scratch_types=[
          pltpu.VMEM((CC // L, L), _f32),
          pltpu.VMEM((CC,), _i32),
          pltpu.VMEM((CC,), _i32),
          pltpu.VMEM((CC,), _i32),
          pltpu.VMEM((CC,), _f32),
      ],
  )(par, zi4)


def _tc1_call(x, slot3, aggx3, degx3, wse, wne, bbe, wsa, wna, bba):
  wspec = lambda s: pl.BlockSpec(s, lambda g: (0,) * len(s))
  return pl.pallas_call(
      _tc1_body,
      grid=(B,),
      in_specs=[
          pl.BlockSpec((1, N, DIN), lambda g: (g, 0, 0)),
          pl.BlockSpec((1, N, N), lambda g: (g, 0, 0)),
          pl.BlockSpec((1, N, DIN), lambda g: (g, 0, 0)),
          pl.BlockSpec((1, N, 1), lambda g: (g, 0, 0)),
          wspec((DIN, DH)), wspec((DIN, DH)), wspec((1, DH)),
          wspec((DIN, DH)), wspec((DIN, DH)), wspec((1, DH)),
      ],
      out_specs=[
          pl.BlockSpec((1, N, DH), lambda g: (g, 0, 0)),
          pl.BlockSpec((1, N, DH), lambda g: (g, 0, 0)),
          pl.BlockSpec((1, N, 1), lambda g: (g, 0, 0)),
      ],
      out_shape=[jax.ShapeDtypeStruct((B, N, DH), _f32),
                 jax.ShapeDtypeStruct((B, N, DH), _f32),
                 jax.ShapeDtypeStruct((B, N, 1), _f32)],
  )(x, slot3, aggx3, degx3, wse, wne, bbe, wsa, wna, bba)


def _tc2_call(h, a0, slot3, aggh3, agga3, deg3, wse, wne, bbe, wsa, wna, bba,
              w1a, w1b):
  wspec = lambda s: pl.BlockSpec(s, lambda g: (0,) * len(s))
  return pl.pallas_call(
      _tc2_body,
      grid=(B,),
      in_specs=[
          pl.BlockSpec((1, N, DH), lambda g: (g, 0, 0)),
          pl.BlockSpec((1, N, DH), lambda g: (g, 0, 0)),
          pl.BlockSpec((1, N, N), lambda g: (g, 0, 0)),
          pl.BlockSpec((1, N, DH), lambda g: (g, 0, 0)),
          pl.BlockSpec((1, N, DH), lambda g: (g, 0, 0)),
          pl.BlockSpec((1, N, 1), lambda g: (g, 0, 0)),
          wspec((DH, DOUT)), wspec((DH, DOUT)), wspec((1, DOUT)),
          wspec((DH, C)), wspec((DH, C)), wspec((1, C)),
          wspec((DOUT, DH)), wspec((DOUT, DH)),
      ],
      out_specs=[
          pl.BlockSpec((1, C, DOUT), lambda g: (g, 0, 0)),
          pl.BlockSpec((1, 1, C), lambda g: (g, 0, 0)),
          pl.BlockSpec((1, C, C), lambda g: (g, 0, 0)),
          pl.BlockSpec((1, N, DH), lambda g: (g, 0, 0)),
          pl.BlockSpec((1, N, DH), lambda g: (g, 0, 0)),
          pl.BlockSpec((1, 1), lambda g: (0, 0)),
      ],
      out_shape=[jax.ShapeDtypeStruct((B, C, DOUT), _f32),
                 jax.ShapeDtypeStruct((B, 1, C), _f32),
                 jax.ShapeDtypeStruct((B, C, C), _f32),
                 jax.ShapeDtypeStruct((B, N, DH), _f32),
                 jax.ShapeDtypeStruct((B, N, DH), _f32),
                 jax.ShapeDtypeStruct((1, 1), _f32)],
  )(h, a0, slot3, aggh3, agga3, deg3, wse, wne, bbe, wsa, wna, bba, w1a, w1b)


def _tc3_call(ug, vg, bb1, w2r, bb2):
  wspec = lambda s: pl.BlockSpec(s, lambda i, j: (0,) * len(s))
  return pl.pallas_call(
      _tc3_body,
      grid=(B, E // EB),
      in_specs=[
          pl.BlockSpec((1, EB, DH), lambda i, j: (i, j, 0)),
          pl.BlockSpec((1, EB, DH), lambda i, j: (i, j, 0)),
          wspec((1, DH)), wspec((1, DH)), wspec((1, 1)),
      ],
      out_specs=pl.BlockSpec((1, 1), lambda i, j: (0, 0)),
      out_shape=jax.ShapeDtypeStruct((1, 1), _f32),
  )(ug, vg, bb1, w2r, bb2)


def kernel(node_features, edge_index, node_mask, edge_mask,
           emb0_Ws, emb0_Wn, emb0_b, emb1_Ws, emb1_Wn, emb1_b,
           asn0_Ws, asn0_Wn, asn0_b, asn1_Ws, asn1_Wn, asn1_b,
           lp_W1, lp_b1, lp_W2, lp_b2):
  x = node_features
  src = edge_index[:, 0, :]
  tgt = edge_index[:, 1, :]
  offs = (jnp.arange(B, dtype=_i32) * N)[:, None]
  srcf = src.reshape(B * E)
  tgtf = tgt.reshape(B * E)
  gsrcf = (src + offs).reshape(B * E)
  gtgtf = (tgt + offs).reshape(B * E)
  x2 = x.reshape(B * N, DIN)

  zi = jnp.zeros((4096,), _i32)
  zf = jnp.zeros((64 * DIN,), _f32)
  zi4 = jnp.zeros((CC,), _i32)

  b0e = emb0_b.reshape(1, DH)
  b0a = asn0_b.reshape(1, DH)
  b1e = emb1_b.reshape(1, DOUT)
  b1a = asn1_b.reshape(1, C)
  w1a = lp_W1[:DOUT]
  w1b = lp_W1[DOUT:]
  bb1 = lp_b1.reshape(1, DH)
  w2r = lp_W2.reshape(1, DH)
  bb2 = lp_b2.reshape(1, 1)

  slot = _sc1a_call(srcf, tgtf, zi)[0]
  exc = _sc1b_call(srcf, tgtf, slot)[0]
  aggx, degx = _sc1c_call(x2, exc, zf)
  slot3 = slot.reshape(B, N, N)
  h, a0, deg3 = _tc1_call(x, slot3, aggx.reshape(B, N, DIN),
                          degx.reshape(B, N, 1),
                          emb0_Ws, emb0_Wn, b0e, asn0_Ws, asn0_Wn, b0a)
  aggh, agga = _sc2_call(h.reshape(B * N, DH), a0.reshape(B * N, DH), exc, zf)
  pooled, pmask, padj, u, v, ent = _tc2_call(
      h, a0, slot3, aggh.reshape(B, N, DH), agga.reshape(B, N, DH), deg3,
      emb1_Ws, emb1_Wn, b1e, asn1_Ws, asn1_Wn, b1a, w1a, w1b)
  ug, vg = _sc3a_call(u.reshape(B * N, DH), v.reshape(B * N, DH),
                      gsrcf, gtgtf)
  pei, pem = _sc3b_call(padj.reshape(B * (CC // L), L), zi4)
  link = _tc3_call(ug.reshape(B, E, DH), vg.reshape(B, E, DH),
                   bb1, w2r, bb2)

  return (pooled, pei.reshape(B, 2, CC), pmask.reshape(B, C),
          pem.reshape(B, CC), link.reshape(()), ent.reshape(()))


# speedup vs baseline: 11.5438x; 1.1197x over previous
import jax
import jax.numpy as jnp
from jax import lax
from jax.experimental import pallas as pl
from jax.experimental.pallas import tpu as pltpu
from jax.experimental.pallas import tpu_sc as plsc

B, N, E, DIN, DH, DOUT, C = 8, 1024, 16384, 256, 256, 256, 64
NC, NS, L = 2, 16, 16
GPC = B // NC
CH = 128
NCH = E // NS // CH
NN = N * N
CC = C * C
ET = E // NS
SLOTT = NN // NS
XCH = 64
XCAP = E // XCH
EB = 2048

_f32 = jnp.float32
_i32 = jnp.int32

_mesh = plsc.VectorSubcoreMesh(
    core_axis_name="c", subcore_axis_name="s", num_cores=NC, num_subcores=NS)
_params = pltpu.CompilerParams(needs_layout_passes=False)


def _sc1a_body(srcf, tgtf, zi,
               slot_o,
               sall, tall, fidx, eidv, zbi):
  cid = lax.axis_index("c")
  sid = lax.axis_index("s")
  pltpu.sync_copy(zi, zbi)
  for gi in range(GPC):
    g = cid * GPC + gi
    for q in range(SLOTT // 16384):
      pltpu.sync_copy(
          zbi, slot_o.at[pl.ds(g * NN + sid * SLOTT + q * 16384, 16384)])
  plsc.subcore_barrier()
  for gi in range(GPC):
    g = cid * GPC + gi
    ebase = g * E + sid * ET
    pltpu.sync_copy(srcf.at[pl.ds(ebase, ET)], sall)
    pltpu.sync_copy(tgtf.at[pl.ds(ebase, ET)], tall)
    for j in range(NCH):
      for k in range(CH // L):
        sv = sall[pl.ds(j * CH + k * L, L)]
        tv = tall[pl.ds(j * CH + k * L, L)]
        fidx[pl.ds(k * L, L)] = sv * N + tv + g * NN
        eidv[pl.ds(k * L, L)] = (
            lax.iota(_i32, L) + (sid * ET + j * CH + k * L + 1))
      pltpu.sync_copy(eidv, slot_o.at[fidx])


def _sc1b_body(srcf, tgtf, slot_i,
               exc_o, cnt_o,
               sall, tall, fidx, rbuf, fbuf, excbuf, cbuf16):
  cid = lax.axis_index("c")
  sid = lax.axis_index("s")
  for gi in range(GPC):
    g = cid * GPC + gi
    ebase = g * E + sid * ET
    pltpu.sync_copy(srcf.at[pl.ds(ebase, ET)], sall)
    pltpu.sync_copy(tgtf.at[pl.ds(ebase, ET)], tall)
    for k in range(ET // L):
      excbuf[pl.ds(k * L, L)] = jnp.full((L,), -1, _i32)
    off = _i32(0)
    for j in range(NCH):
      for k in range(CH // L):
        sv = sall[pl.ds(j * CH + k * L, L)]
        tv = tall[pl.ds(j * CH + k * L, L)]
        f = sv * N + tv
        fbuf[pl.ds(k * L, L)] = f
        fidx[pl.ds(k * L, L)] = f + g * NN
      pltpu.sync_copy(slot_i.at[fidx], rbuf)
      for k in range(CH // L):
        rv = rbuf[pl.ds(k * L, L)]
        ev = lax.iota(_i32, L) + (sid * ET + j * CH + k * L + 1)
        m = rv != ev
        mi = m.astype(_i32)
        pos = plsc.cumsum(mi) + (off - 1)
        plsc.store_scatter(excbuf, [pos], fbuf[pl.ds(k * L, L)], mask=m)
        off = off + jnp.sum(mi)
    pltpu.sync_copy(excbuf, exc_o.at[pl.ds((g * NS + sid) * ET, ET)])
    cbuf16[pl.ds(0, L)] = jnp.full((L,), off, _i32)
    pltpu.sync_copy(cbuf16, cnt_o.at[pl.ds((g * NS + sid) * L, L)])


def _sc1c_body(x2, exc_i, cnt_i, zf,
               aggx_o, degx_o,
               excall, cntb, own2d, tl2d, tlbuf, gbuf, stripe, degst):
  cid = lax.axis_index("c")
  sid = lax.axis_index("s")
  for gi in range(GPC):
    g = cid * GPC + gi
    pltpu.sync_copy(exc_i.at[pl.ds(g * NS * ET, NS * ET)], excall)
    pltpu.sync_copy(cnt_i.at[pl.ds(g * NS * L, NS * L)], cntb)
    pltpu.sync_copy(zf, stripe)
    for k in range(4):
      degst[pl.ds(k * L, L)] = jnp.zeros((L,), _f32)
    nmat = _i32(0)
    for q in range(NS):
      nq = cntb[pl.ds(q * L, L)][0]

      def scan_body(k, nmat, q=q):
        pv = excall[pl.ds(q * ET + k * L, L)]
        t = jnp.bitwise_and(pv, N - 1)
        m = jnp.logical_and(pv >= 0, lax.shift_right_logical(t, 6) == sid)
        mi = m.astype(_i32)
        pos = plsc.cumsum(mi) + (nmat - 1)
        grow = lax.shift_right_logical(pv, 10) + g * N
        tl = jnp.bitwise_and(pv, 63)
        r = lax.shift_right_logical(pos, 6)
        cc = jnp.bitwise_and(pos, XCH - 1)
        plsc.store_scatter(own2d, [r, cc], grow, mask=m)
        plsc.store_scatter(tl2d, [r, cc], tl, mask=m)
        return nmat + jnp.sum(mi)

      nmat = lax.fori_loop(0, lax.shift_right_logical(nq + (L - 1), 4),
                           scan_body, nmat)
    rlast = jnp.minimum(lax.shift_right_logical(nmat, 6), XCAP - 1)
    full_cap = nmat >= XCAP * XCH
    nrem = jnp.bitwise_and(nmat, XCH - 1)
    for k in range(XCH // L):
      v = own2d[rlast, pl.ds(k * L, L)]
      keep = jnp.logical_or((lax.iota(_i32, L) + (k * L)) < nrem, full_cap)
      vv = jnp.where(keep, v, g * N)
      plsc.store_scatter(own2d,
                         [jnp.full((L,), rlast, _i32),
                          lax.iota(_i32, L) + (k * L)], vv)
    nch = lax.shift_right_logical(nmat + (XCH - 1), 6)

    def chunk_body(c, acc):
      pltpu.sync_copy(x2.at[own2d.at[c]], gbuf)
      for k in range(XCH // L):
        tlbuf[pl.ds(k * L, L)] = tl2d[c, pl.ds(k * L, L)]
      nin = jnp.minimum(nmat - c * XCH, XCH)

      def ebody(e, a2):
        tl = jnp.bitwise_and(tlbuf[pl.ds(e, L)][0], 63)
        for k2 in range(DIN // L):
          xv = gbuf[e, pl.ds(k2 * L, L)]
          idxv = tl * DIN + (k2 * L) + lax.iota(_i32, L)
          plsc.addupdate_scatter(stripe, [idxv], xv)
        onemask = lax.iota(_i32, L) == 0
        plsc.addupdate_scatter(degst, [jnp.full((L,), tl, _i32)],
                               jnp.ones((L,), _f32), mask=onemask)
        return a2

      return lax.fori_loop(0, nin, ebody, acc)

    lax.fori_loop(0, nch, chunk_body, _i32(0))
    pltpu.sync_copy(stripe,
                    aggx_o.at[pl.ds(g * N * DIN + sid * 64 * DIN, 64 * DIN)])
    pltpu.sync_copy(degst.at[pl.ds(0, 64)],
                    degx_o.at[pl.ds(g * N + sid * 64, 64)])


def _sc2_body(h2, a2, exc_i, cnt_i, zf,
              aggh_o, agga_o,
              excall, cntb, own2d, tl2d, tlbuf, gbuf, strh, stra):
  cid = lax.axis_index("c")
  sid = lax.axis_index("s")
  for gi in range(GPC):
    g = cid * GPC + gi
    pltpu.sync_copy(cnt_i.at[pl.ds(g * NS * L, NS * L)], cntb)
    pltpu.sync_copy(zf, strh)
    pltpu.sync_copy(zf, stra)
    nmat = _i32(0)
    for half in range(2):
      pltpu.sync_copy(
          exc_i.at[pl.ds(g * NS * ET + half * (NS // 2) * ET, NS // 2 * ET)],
          excall)
      for q in range(NS // 2):
        nq = cntb[pl.ds((half * (NS // 2) + q) * L, L)][0]

        def scan_body(k, nmat, q=q):
          pv = excall[pl.ds(q * ET + k * L, L)]
          t = jnp.bitwise_and(pv, N - 1)
          m = jnp.logical_and(pv >= 0, lax.shift_right_logical(t, 6) == sid)
          mi = m.astype(_i32)
          pos = plsc.cumsum(mi) + (nmat - 1)
          grow = lax.shift_right_logical(pv, 10) + g * N
          tl = jnp.bitwise_and(pv, 63)
          r = lax.shift_right_logical(pos, 6)
          cc = jnp.bitwise_and(pos, XCH - 1)
          plsc.store_scatter(own2d, [r, cc], grow, mask=m)
          plsc.store_scatter(tl2d, [r, cc], tl, mask=m)
          return nmat + jnp.sum(mi)

        nmat = lax.fori_loop(0, lax.shift_right_logical(nq + (L - 1), 4),
                             scan_body, nmat)
    rlast = jnp.minimum(lax.shift_right_logical(nmat, 6), XCAP - 1)
    full_cap = nmat >= XCAP * XCH
    nrem = jnp.bitwise_and(nmat, XCH - 1)
    for k in range(XCH // L):
      v = own2d[rlast, pl.ds(k * L, L)]
      keep = jnp.logical_or((lax.iota(_i32, L) + (k * L)) < nrem, full_cap)
      vv = jnp.where(keep, v, g * N)
      plsc.store_scatter(own2d,
                         [jnp.full((L,), rlast, _i32),
                          lax.iota(_i32, L) + (k * L)], vv)
    nch = lax.shift_right_logical(nmat + (XCH - 1), 6)

    def chunk_body(c, acc):
      for k in range(XCH // L):
        tlbuf[pl.ds(k * L, L)] = tl2d[c, pl.ds(k * L, L)]
      nin = jnp.minimum(nmat - c * XCH, XCH)
      pltpu.sync_copy(h2.at[own2d.at[c]], gbuf)

      def ebody(e, a2c):
        tl = jnp.bitwise_and(tlbuf[pl.ds(e, L)][0], 63)
        for k2 in range(DH // L):
          idxv = tl * DH + (k2 * L) + lax.iota(_i32, L)
          plsc.addupdate_scatter(strh, [idxv], gbuf[e, pl.ds(k2 * L, L)])
        return a2c

      lax.fori_loop(0, nin, ebody, acc)
      pltpu.sync_copy(a2.at[own2d.at[c]], gbuf)

      def ebody2(e, a2c):
        tl = jnp.bitwise_and(tlbuf[pl.ds(e, L)][0], 63)
        for k2 in range(DH // L):
          idxv = tl * DH + (k2 * L) + lax.iota(_i32, L)
          plsc.addupdate_scatter(stra, [idxv], gbuf[e, pl.ds(k2 * L, L)])
        return a2c

      return lax.fori_loop(0, nin, ebody2, acc)

    lax.fori_loop(0, nch, chunk_body, _i32(0))
    pltpu.sync_copy(strh,
                    aggh_o.at[pl.ds(g * N * DH + sid * 64 * DH, 64 * DH)])
    pltpu.sync_copy(stra,
                    agga_o.at[pl.ds(g * N * DH + sid * 64 * DH, 64 * DH)])


def _sc3a_body(u2, v2, gsrcf, gtgtf,
               ug_o, vg_o,
               six, tix, gbuf):
  cid = lax.axis_index("c")
  sid = lax.axis_index("s")
  for gi in range(GPC):
    g = cid * GPC + gi
    ebase = g * E + sid * ET
    for j in range(NCH):
      pltpu.sync_copy(gsrcf.at[pl.ds(ebase + j * CH, CH)], six)
      pltpu.sync_copy(gtgtf.at[pl.ds(ebase + j * CH, CH)], tix)
      pltpu.sync_copy(u2.at[six], gbuf)
      pltpu.sync_copy(gbuf, ug_o.at[pl.ds(ebase + j * CH, CH)])
      pltpu.sync_copy(v2.at[tix], gbuf)
      pltpu.sync_copy(gbuf, vg_o.at[pl.ds(ebase + j * CH, CH)])


def _sc3b_body(par, zi4,
               pei_o, pem_o,
               pav, fbuf, sbuf, tbuf, mbuf):
  cid = lax.axis_index("c")
  sid = lax.axis_index("s")

  @pl.when(sid < GPC)
  def _():
    g = cid * GPC + sid
    pltpu.sync_copy(par.at[pl.ds(g * (CC // L), CC // L)], pav)
    pltpu.sync_copy(zi4, fbuf)
    off = _i32(0)
    for k in range(CC // L):
      pa = pav[k]
      m = pa > 1e-6
      mi = m.astype(_i32)
      fv = lax.iota(_i32, L) + (k * L)
      pos = plsc.cumsum(mi) + (off - 1)
      plsc.store_scatter(fbuf, [pos], fv, mask=m)
      off = off + jnp.sum(mi)
    for k in range(CC // L):
      fv = fbuf[pl.ds(k * L, L)]
      sbuf[pl.ds(k * L, L)] = lax.shift_right_logical(fv, 6)
      tbuf[pl.ds(k * L, L)] = jnp.bitwise_and(fv, C - 1)
      inr = (lax.iota(_i32, L) + (k * L)) < off
      mbuf[pl.ds(k * L, L)] = jnp.where(inr, _f32(1.0), _f32(0.0))
    pltpu.sync_copy(sbuf, pei_o.at[pl.ds((g * 2) * CC, CC)])
    pltpu.sync_copy(tbuf, pei_o.at[pl.ds((g * 2 + 1) * CC, CC)])
    pltpu.sync_copy(mbuf, pem_o.at[pl.ds(g * CC, CC)])


def _tc1_body(x_ref, slot_ref, aggx_ref, degx_ref,
              wse, wne, bbe, wsa, wna, bba,
              h_ref, a0_ref, deg_ref):
  x = x_ref[0]
  adj = (slot_ref[0] > 0).astype(_f32)
  agg = lax.dot_general(adj, x, (((0,), (0,)), ((), ())),
                        preferred_element_type=_f32)
  agg += aggx_ref[0]
  deg = lax.dot_general(adj, jnp.ones((N, 1), _f32), (((0,), (0,)), ((), ())),
                        preferred_element_type=_f32) + degx_ref[0]
  deg_ref[0] = deg
  degc = jnp.maximum(deg, 1e-8)
  aggn = agg / degc
  h = jnp.dot(x, wse[...], preferred_element_type=_f32)
  h += jnp.dot(aggn, wne[...], preferred_element_type=_f32)
  h_ref[0] = jnp.maximum(h + bbe[...], 0.0)
  a = jnp.dot(x, wsa[...], preferred_element_type=_f32)
  a += jnp.dot(aggn, wna[...], preferred_element_type=_f32)
  a0_ref[0] = jnp.maximum(a + bba[...], 0.0)


def _tc2_body(h_ref, a0_ref, slot_ref, aggh_ref, agga_ref, deg_ref,
              wse, wne, bbe, wsa, wna, bba, w1a, w1b,
              pooled_ref, pmask_ref, padj_ref, u_ref, v_ref, ent_ref):
  g = pl.program_id(0)
  adj = (slot_ref[0] > 0).astype(_f32)
  degc = jnp.maximum(deg_ref[0], 1e-8)
  aggh = lax.dot_general(adj, h_ref[0], (((0,), (0,)), ((), ())),
                         preferred_element_type=_f32) + aggh_ref[0]
  agga = lax.dot_general(adj, a0_ref[0], (((0,), (0,)), ((), ())),
                         preferred_element_type=_f32) + agga_ref[0]
  emb = jnp.dot(h_ref[0], wse[...], preferred_element_type=_f32)
  emb += jnp.dot(aggh / degc, wne[...], preferred_element_type=_f32)
  emb = jnp.maximum(emb + bbe[...], 0.0)
  a1 = jnp.dot(a0_ref[0], wsa[...], preferred_element_type=_f32)
  a1 += jnp.dot(agga / degc, wna[...], preferred_element_type=_f32)
  a1 = jnp.maximum(a1 + bba[...], 0.0)
  mx = jnp.max(a1, axis=-1, keepdims=True)
  ex = jnp.exp(a1 - mx)
  S = ex / jnp.sum(ex, axis=-1, keepdims=True)
  pooled_ref[0] = lax.dot_general(S, emb, (((0,), (0,)), ((), ())),
                                  preferred_element_type=_f32)
  colsum = jnp.sum(S, axis=0, keepdims=True)
  pmask_ref[0] = (colsum > 1e-6).astype(_f32)
  t1 = lax.dot_general(S, adj, (((0,), (0,)), ((), ())),
                       preferred_element_type=_f32)
  padj_ref[0] = jnp.dot(t1, S, preferred_element_type=_f32)
  u_ref[0] = jnp.dot(emb, w1a[...], preferred_element_type=_f32)
  v_ref[0] = jnp.dot(emb, w1b[...], preferred_element_type=_f32)
  ent = -jnp.sum(S * jnp.log(jnp.maximum(S, 1e-8)), axis=-1, keepdims=True)

  @pl.when(g == 0)
  def _():
    ent_ref[...] = jnp.zeros((1, 1), _f32)

  ent_ref[...] += jnp.sum(ent).reshape(1, 1) / (N * B)


def _tc3_body(u_ref, v_ref, bb1, w2r, bb2, out_ref):
  i = pl.program_id(0)
  j = pl.program_id(1)
  w = jnp.maximum(u_ref[0] + v_ref[0] + bb1[...], 0.0)
  z = jnp.sum(w * w2r[...], axis=-1, keepdims=True) + bb2[...]
  p = jnp.clip(jax.nn.sigmoid(z), 1e-7, 1.0 - 1e-7)
  loss = jnp.sum(-jnp.log(p))

  @pl.when(jnp.logical_and(i == 0, j == 0))
  def _():
    out_ref[...] = jnp.zeros((1, 1), _f32)

  out_ref[...] += loss.reshape(1, 1) / (E * B)


def _sc1a_call(srcf, tgtf, zi):
  return pl.kernel(
      _sc1a_body,
      out_type=[jax.ShapeDtypeStruct((B * NN,), _i32)],
      mesh=_mesh,
      compiler_params=_params,
      scratch_types=[
          pltpu.VMEM((ET,), _i32),
          pltpu.VMEM((ET,), _i32),
          pltpu.VMEM((CH,), _i32),
          pltpu.VMEM((CH,), _i32),
          pltpu.VMEM((16384,), _i32),
      ],
  )(srcf, tgtf, zi)


def _sc1b_call(srcf, tgtf, slot):
  return pl.kernel(
      _sc1b_body,
      out_type=[jax.ShapeDtypeStruct((B * E,), _i32),
                jax.ShapeDtypeStruct((B * NS * L,), _i32)],
      mesh=_mesh,
      compiler_params=_params,
      scratch_types=[
          pltpu.VMEM((ET,), _i32),
          pltpu.VMEM((ET,), _i32),
          pltpu.VMEM((CH,), _i32),
          pltpu.VMEM((CH,), _i32),
          pltpu.VMEM((CH,), _i32),
          pltpu.VMEM((ET,), _i32),
          pltpu.VMEM((L,), _i32),
      ],
  )(srcf, tgtf, slot)


def _sc1c_call(x2, exc, cnt, zf):
  return pl.kernel(
      _sc1c_body,
      out_type=[jax.ShapeDtypeStruct((B * N * DIN,), _f32),
                jax.ShapeDtypeStruct((B * N,), _f32)],
      mesh=_mesh,
      compiler_params=_params,
      scratch_types=[
          pltpu.VMEM((NS * ET,), _i32),
          pltpu.VMEM((NS * L,), _i32),
          pltpu.VMEM((XCAP, XCH), _i32),
          pltpu.VMEM((XCAP, XCH), _i32),
          pltpu.VMEM((XCH + L,), _i32),
          pltpu.VMEM((XCH, DIN), _f32),
          pltpu.VMEM((64 * DIN,), _f32),
          pltpu.VMEM((64,), _f32),
      ],
  )(x2, exc, cnt, zf)


def _sc2_call(h2, a2, exc, cnt, zf):
  return pl.kernel(
      _sc2_body,
      out_type=[jax.ShapeDtypeStruct((B * N * DH,), _f32),
                jax.ShapeDtypeStruct((B * N * DH,), _f32)],
      mesh=_mesh,
      compiler_params=_params,
      scratch_types=[
          pltpu.VMEM((NS // 2 * ET,), _i32),
          pltpu.VMEM((NS * L,), _i32),
          pltpu.VMEM((XCAP, XCH), _i32),
          pltpu.VMEM((XCAP, XCH), _i32),
          pltpu.VMEM((XCH + L,), _i32),
          pltpu.VMEM((XCH, DH), _f32),
          pltpu.VMEM((64 * DH,), _f32),
          pltpu.VMEM((64 * DH,), _f32),
      ],
  )(h2, a2, exc, cnt, zf)


def _sc3a_call(u2, v2, gsrcf, gtgtf):
  return pl.kernel(
      _sc3a_body,
      out_type=[jax.ShapeDtypeStruct((B * E, DH), _f32),
                jax.ShapeDtypeStruct((B * E, DH), _f32)],
      mesh=_mesh,
      compiler_params=_params,
      scratch_types=[
          pltpu.VMEM((CH,), _i32),
          pltpu.VMEM((CH,), _i32),
          pltpu.VMEM((CH, DH), _f32),
      ],
  )(u2, v2, gsrcf, gtgtf)


def _sc3b_call(par, zi4):
  return pl.kernel(
      _sc3b_body,
      out_type=[jax.ShapeDtypeStruct((B * 2 * CC,), _i32),
                jax.ShapeDtypeStruct((B * CC,), _f32)],
      mesh=_mesh,
      compiler_params=_params,
      scratch_types=[
          pltpu.VMEM((CC // L, L), _f32),
          pltpu.VMEM((CC,), _i32),
          pltpu.VMEM((CC,), _i32),
          pltpu.VMEM((CC,), _i32),
          pltpu.VMEM((CC,), _f32),
      ],
  )(par, zi4)


def _tc1_call(x, slot3, aggx3, degx3, wse, wne, bbe, wsa, wna, bba):
  wspec = lambda s: pl.BlockSpec(s, lambda g: (0,) * len(s))
  return pl.pallas_call(
      _tc1_body,
      grid=(B,),
      in_specs=[
          pl.BlockSpec((1, N, DIN), lambda g: (g, 0, 0)),
          pl.BlockSpec((1, N, N), lambda g: (g, 0, 0)),
          pl.BlockSpec((1, N, DIN), lambda g: (g, 0, 0)),
          pl.BlockSpec((1, N, 1), lambda g: (g, 0, 0)),
          wspec((DIN, DH)), wspec((DIN, DH)), wspec((1, DH)),
          wspec((DIN, DH)), wspec((DIN, DH)), wspec((1, DH)),
      ],
      out_specs=[
          pl.BlockSpec((1, N, DH), lambda g: (g, 0, 0)),
          pl.BlockSpec((1, N, DH), lambda g: (g, 0, 0)),
          pl.BlockSpec((1, N, 1), lambda g: (g, 0, 0)),
      ],
      out_shape=[jax.ShapeDtypeStruct((B, N, DH), _f32),
                 jax.ShapeDtypeStruct((B, N, DH), _f32),
                 jax.ShapeDtypeStruct((B, N, 1), _f32)],
  )(x, slot3, aggx3, degx3, wse, wne, bbe, wsa, wna, bba)


def _tc2_call(h, a0, slot3, aggh3, agga3, deg3, wse, wne, bbe, wsa, wna, bba,
              w1a, w1b):
  wspec = lambda s: pl.BlockSpec(s, lambda g: (0,) * len(s))
  return pl.pallas_call(
      _tc2_body,
      grid=(B,),
      in_specs=[
          pl.BlockSpec((1, N, DH), lambda g: (g, 0, 0)),
          pl.BlockSpec((1, N, DH), lambda g: (g, 0, 0)),
          pl.BlockSpec((1, N, N), lambda g: (g, 0, 0)),
          pl.BlockSpec((1, N, DH), lambda g: (g, 0, 0)),
          pl.BlockSpec((1, N, DH), lambda g: (g, 0, 0)),
          pl.BlockSpec((1, N, 1), lambda g: (g, 0, 0)),
          wspec((DH, DOUT)), wspec((DH, DOUT)), wspec((1, DOUT)),
          wspec((DH, C)), wspec((DH, C)), wspec((1, C)),
          wspec((DOUT, DH)), wspec((DOUT, DH)),
      ],
      out_specs=[
          pl.BlockSpec((1, C, DOUT), lambda g: (g, 0, 0)),
          pl.BlockSpec((1, 1, C), lambda g: (g, 0, 0)),
          pl.BlockSpec((1, C, C), lambda g: (g, 0, 0)),
          pl.BlockSpec((1, N, DH), lambda g: (g, 0, 0)),
          pl.BlockSpec((1, N, DH), lambda g: (g, 0, 0)),
          pl.BlockSpec((1, 1), lambda g: (0, 0)),
      ],
      out_shape=[jax.ShapeDtypeStruct((B, C, DOUT), _f32),
                 jax.ShapeDtypeStruct((B, 1, C), _f32),
                 jax.ShapeDtypeStruct((B, C, C), _f32),
                 jax.ShapeDtypeStruct((B, N, DH), _f32),
                 jax.ShapeDtypeStruct((B, N, DH), _f32),
                 jax.ShapeDtypeStruct((1, 1), _f32)],
  )(h, a0, slot3, aggh3, agga3, deg3, wse, wne, bbe, wsa, wna, bba, w1a, w1b)


def _tc3_call(ug, vg, bb1, w2r, bb2):
  wspec = lambda s: pl.BlockSpec(s, lambda i, j: (0,) * len(s))
  return pl.pallas_call(
      _tc3_body,
      grid=(B, E // EB),
      in_specs=[
          pl.BlockSpec((1, EB, DH), lambda i, j: (i, j, 0)),
          pl.BlockSpec((1, EB, DH), lambda i, j: (i, j, 0)),
          wspec((1, DH)), wspec((1, DH)), wspec((1, 1)),
      ],
      out_specs=pl.BlockSpec((1, 1), lambda i, j: (0, 0)),
      out_shape=jax.ShapeDtypeStruct((1, 1), _f32),
  )(ug, vg, bb1, w2r, bb2)


def kernel(node_features, edge_index, node_mask, edge_mask,
           emb0_Ws, emb0_Wn, emb0_b, emb1_Ws, emb1_Wn, emb1_b,
           asn0_Ws, asn0_Wn, asn0_b, asn1_Ws, asn1_Wn, asn1_b,
           lp_W1, lp_b1, lp_W2, lp_b2):
  x = node_features
  src = edge_index[:, 0, :]
  tgt = edge_index[:, 1, :]
  offs = (jnp.arange(B, dtype=_i32) * N)[:, None]
  srcf = src.reshape(B * E)
  tgtf = tgt.reshape(B * E)
  gsrcf = (src + offs).reshape(B * E)
  gtgtf = (tgt + offs).reshape(B * E)
  x2 = x.reshape(B * N, DIN)

  zi = jnp.zeros((16384,), _i32)
  zf = jnp.zeros((64 * DIN,), _f32)
  zi4 = jnp.zeros((CC,), _i32)

  b0e = emb0_b.reshape(1, DH)
  b0a = asn0_b.reshape(1, DH)
  b1e = emb1_b.reshape(1, DOUT)
  b1a = asn1_b.reshape(1, C)
  w1a = lp_W1[:DOUT]
  w1b = lp_W1[DOUT:]
  bb1 = lp_b1.reshape(1, DH)
  w2r = lp_W2.reshape(1, DH)
  bb2 = lp_b2.reshape(1, 1)

  slot = _sc1a_call(srcf, tgtf, zi)[0]
  exc, cnt = _sc1b_call(srcf, tgtf, slot)
  aggx, degx = _sc1c_call(x2, exc, cnt, zf)
  slot3 = slot.reshape(B, N, N)
  h, a0, deg3 = _tc1_call(x, slot3, aggx.reshape(B, N, DIN),
                          degx.reshape(B, N, 1),
                          emb0_Ws, emb0_Wn, b0e, asn0_Ws, asn0_Wn, b0a)
  aggh, agga = _sc2_call(h.reshape(B * N, DH), a0.reshape(B * N, DH),
                         exc, cnt, zf)
  pooled, pmask, padj, u, v, ent = _tc2_call(
      h, a0, slot3, aggh.reshape(B, N, DH), agga.reshape(B, N, DH), deg3,
      emb1_Ws, emb1_Wn, b1e, asn1_Ws, asn1_Wn, b1a, w1a, w1b)
  ug, vg = _sc3a_call(u.reshape(B * N, DH), v.reshape(B * N, DH),
                      gsrcf, gtgtf)
  pei, pem = _sc3b_call(padj.reshape(B * (CC // L), L), zi4)
  link = _tc3_call(ug.reshape(B, E, DH), vg.reshape(B, E, DH),
                   bb1, w2r, bb2)

  return (pooled, pei.reshape(B, 2, CC), pmask.reshape(B, C),
          pem.reshape(B, CC), link.reshape(()), ent.reshape(()))
